# Initial kernel scaffold; baseline (speedup 1.0000x reference)
import jax
import jax.numpy as jnp
from jax.experimental import pallas as pl


def kernel(x, edge_index, W, b, gamma, beta):
    raise NotImplementedError("write your pallas kernel here")


# speedup vs baseline: 5.4552x; 5.4552x over previous
import functools

import jax
import jax.numpy as jnp
from jax import lax
from jax.experimental import pallas as pl
from jax.experimental.pallas import tpu as pltpu
from jax.experimental.pallas import tpu_sc as plsc

N = 10000
D = 128
E = 320000

NC = 2
NS = 16
NW = NC * NS

K = 128
CH = 79
EPW = K * CH
E_PAD = NW * EPW
NR = 10240
RPT = NR // NS
DW = 16

BLK = 1024
GRID = NR // BLK

_mesh = plsc.VectorSubcoreMesh(
    core_axis_name="c", subcore_axis_name="s", num_cores=NC, num_subcores=NS
)


def _deg_body(src_hbm, dst_hbm, out, cnt_src, cnt_dst, idx_v):
    c = lax.axis_index("c")
    s = lax.axis_index("s")
    w = c * NS + s
    base = w * EPW
    ones16 = jnp.ones((16,), jnp.float32)
    zeros16 = jnp.zeros((16,), jnp.float32)

    @pl.loop(0, NR // 16)
    def _zero(i):
        cnt_src[pl.ds(i * 16, 16)] = zeros16
        cnt_dst[pl.ds(i * 16, 16)] = zeros16

    @pl.loop(0, CH)
    def _edges(j):
        off = base + j * K
        pltpu.sync_copy(src_hbm.at[pl.ds(off, K)], idx_v)
        for q in range(K // 16):
            plsc.addupdate_scatter(cnt_src, [idx_v[pl.ds(q * 16, 16)]], ones16)
        pltpu.sync_copy(dst_hbm.at[pl.ds(off, K)], idx_v)
        for q in range(K // 16):
            plsc.addupdate_scatter(cnt_dst, [idx_v[pl.ds(q * 16, 16)]], ones16)

    pltpu.sync_copy(cnt_src, out.at[0, w])
    pltpu.sync_copy(cnt_dst, out.at[1, w])


def _agg_body(xs_hbm, src_hbm, dst_hbm, out_hbm,
              acc, sidx_v, didx_v, rows_v, zero_v, sem):
    c = lax.axis_index("c")
    s = lax.axis_index("s")
    base = (c * NS + s) * EPW

    @pl.loop(0, K)
    def _fill(i):
        for q in range(D // 16):
            zero_v[i, pl.ds(q * 16, 16)] = jnp.zeros((16,), jnp.float32)

    @pl.loop(0, RPT // K)
    def _zero(i):
        pltpu.sync_copy(zero_v, acc.at[pl.ds(s * RPT + i * K, K)])

    plsc.subcore_barrier()

    @pl.loop(0, CH)
    def _edges(j):
        off = base + j * K
        pltpu.sync_copy(src_hbm.at[pl.ds(off, K)], sidx_v)
        pltpu.sync_copy(dst_hbm.at[pl.ds(off, K)], didx_v)
        pltpu.async_copy(xs_hbm.at[sidx_v], rows_v, sem).wait()
        pltpu.sync_copy(rows_v, acc.at[didx_v], add=True)

    plsc.subcore_barrier()

    row = s * RPT
    pltpu.sync_copy(acc.at[pl.ds(row, RPT)], out_hbm.at[c, pl.ds(row, RPT)])


def _make_deg_kernel(interpret=False):
    return pl.kernel(
        _deg_body,
        out_type=jax.ShapeDtypeStruct((2, NW, NR), jnp.float32),
        mesh=_mesh,
        scratch_types=[
            pltpu.VMEM((NR,), jnp.float32),
            pltpu.VMEM((NR,), jnp.float32),
            pltpu.VMEM((K,), jnp.int32),
        ],
        compiler_params=pltpu.CompilerParams(needs_layout_passes=False),
        interpret=interpret,
    )


def _make_agg_kernel(interpret=False):
    return pl.kernel(
        _agg_body,
        out_type=jax.ShapeDtypeStruct((NC, NR, D), jnp.float32),
        mesh=_mesh,
        scratch_types=[
            pltpu.VMEM_SHARED((NR, D), jnp.float32),
            pltpu.VMEM((K,), jnp.int32),
            pltpu.VMEM((K,), jnp.int32),
            pltpu.VMEM((K, D), jnp.float32),
            pltpu.VMEM((K, D), jnp.float32),
            pltpu.SemaphoreType.DMA,
        ],
        interpret=interpret,
    )


_deg_kernel = _make_deg_kernel()
_agg_kernel = _make_agg_kernel()


def _scale_body(x_ref, deg_ref, o_ref):
    cnt = jnp.maximum(jnp.sum(deg_ref[...], axis=0), 1.0)
    o_ref[...] = x_ref[...] * lax.rsqrt(cnt)


def _scale_call(x_pad, deg_src):
    return pl.pallas_call(
        _scale_body,
        grid=(GRID,),
        in_specs=[
            pl.BlockSpec((BLK, D), lambda i: (i, 0)),
            pl.BlockSpec((NW, BLK, 1), lambda i: (0, i, 0)),
        ],
        out_specs=pl.BlockSpec((BLK, D), lambda i: (i, 0)),
        out_shape=jax.ShapeDtypeStruct((NR, D), jnp.float32),
    )(x_pad, deg_src)


_BN_INV = 1.0 / (1.0 + 1e-5) ** 0.5


def _final_body(agg_ref, deg_ref, x_ref, w_ref, b_ref, g_ref, bt_ref, o_ref):
    a = agg_ref[0] + agg_ref[1]
    nd = lax.rsqrt(jnp.maximum(jnp.sum(deg_ref[...], axis=0), 1.0))
    z = jnp.dot(a, w_ref[...], preferred_element_type=jnp.float32)
    z = z * nd + b_ref[...] + x_ref[...]
    z = z * (g_ref[...] * _BN_INV) + bt_ref[...]
    o_ref[...] = jnp.maximum(z, 0.0)


def _final_call(agg, deg_dst, x_pad, W, b2, g2, bt2):
    return pl.pallas_call(
        _final_body,
        grid=(GRID,),
        in_specs=[
            pl.BlockSpec((NC, BLK, D), lambda i: (0, i, 0)),
            pl.BlockSpec((NW, BLK, 1), lambda i: (0, i, 0)),
            pl.BlockSpec((BLK, D), lambda i: (i, 0)),
            pl.BlockSpec((D, D), lambda i: (0, 0)),
            pl.BlockSpec((1, D), lambda i: (0, 0)),
            pl.BlockSpec((1, D), lambda i: (0, 0)),
            pl.BlockSpec((1, D), lambda i: (0, 0)),
        ],
        out_specs=pl.BlockSpec((BLK, D), lambda i: (i, 0)),
        out_shape=jax.ShapeDtypeStruct((NR, D), jnp.float32),
    )(agg, deg_dst, x_pad, W, b2, g2, bt2)


def kernel(x, edge_index, W, b, gamma, beta):
    src = edge_index[0]
    dst = edge_index[1]
    pad = jnp.full((E_PAD - E,), N, dtype=jnp.int32)
    srcp = jnp.concatenate([src, pad])
    dstp = jnp.concatenate([dst, pad])
    x_pad = jnp.pad(x, ((0, NR - N), (0, 0)))

    degs = _deg_kernel(srcp, dstp)
    deg_src = degs[0].reshape(NW, NR, 1)
    deg_dst = degs[1].reshape(NW, NR, 1)
    xs = _scale_call(x_pad, deg_src)
    agg = _agg_kernel(xs, srcp, dstp)
    out = _final_call(agg, deg_dst, x_pad, W,
                      b.reshape(1, D), gamma.reshape(1, D), beta.reshape(1, D))
    return out[:N]
